# baseline (speedup 1.0000x reference)
import functools

import jax
import jax.numpy as jnp
from jax import lax
from jax.experimental import pallas as pl
from jax.experimental.pallas import tpu as pltpu
from jax.experimental.pallas import tpu_sc as plsc

B = 16384
NUM_CONT = 13
NUM_CAT = 26
VOCAB = 100001
EMB = 32
H_CONT = 32
H1 = 64
H2 = 32

N = B * NUM_CAT
NW = 32
PER_W = N // NW
G = 128
NG = PER_W // G
KBUF = 8
NOUT = NG // KBUF


def _make_sc_gather():
    mesh = plsc.VectorSubcoreMesh(core_axis_name="c", subcore_axis_name="s")

    @functools.partial(
        pl.kernel,
        mesh=mesh,
        out_type=jax.ShapeDtypeStruct((N, EMB), jnp.float32),
        scratch_types=[
            pltpu.VMEM((NG, G), jnp.int32),
            pltpu.VMEM((KBUF * G, EMB), jnp.float32),
            pltpu.SemaphoreType.DMA,
        ],
        compiler_params=pltpu.CompilerParams(use_tc_tiling_on_sc=False),
    )
    def gather_k(idx_hbm, table_hbm, out_hbm, idx_v, rows_v, sem):
        wid = lax.axis_index("s") * 2 + lax.axis_index("c")
        pltpu.sync_copy(idx_hbm.at[wid], idx_v)

        def outer(i, carry):
            handles = []
            for b in range(KBUF):
                g = i * KBUF + b
                h = pltpu.make_async_copy(
                    table_hbm.at[idx_v.at[g]],
                    rows_v.at[pl.ds(b * G, G)],
                    sem,
                )
                h.start()
                handles.append(h)
            for h in handles:
                h.wait()
            base = wid * PER_W + i * (KBUF * G)
            pltpu.sync_copy(rows_v, out_hbm.at[pl.ds(base, KBUF * G)])
            return carry

        lax.fori_loop(0, NOUT, outer, 0)

    return gather_k


_sc_gather = _make_sc_gather()


def _mlp_body(xc_ref, cat_ref, w1t_ref, b1_ref, g_ref, be_ref,
              w2ht_ref, w2ct_ref, b2_ref, w3t_ref, b3_ref, out_ref):
    h = jnp.dot(xc_ref[...], w1t_ref[...],
                preferred_element_type=jnp.float32) + b1_ref[...]
    mu = jnp.mean(h, axis=-1, keepdims=True)
    var = jnp.mean((h - mu) * (h - mu), axis=-1, keepdims=True)
    h = (h - mu) * lax.rsqrt(var + 1e-5) * g_ref[...] + be_ref[...]
    h = h * (1.0 / (1.0 + jnp.exp(-h)))
    z = (jnp.dot(h, w2ht_ref[...], preferred_element_type=jnp.float32)
         + jnp.dot(cat_ref[...], w2ct_ref[...],
                   preferred_element_type=jnp.float32)
         + b2_ref[...])
    z = z * (1.0 / (1.0 + jnp.exp(-z)))
    z = jnp.dot(z, w3t_ref[...], preferred_element_type=jnp.float32) + b3_ref[...]
    out_ref[...] = z * (1.0 / (1.0 + jnp.exp(-z)))


_ROWS = 1024


def _mlp_call(x_cont, cat, w1t, b1, ln_g, ln_b, w2ht, w2ct, b2, w3t, b3):
    grid = (B // _ROWS,)
    full = lambda *shape: pl.BlockSpec(shape, lambda i: (0,) * len(shape))
    return pl.pallas_call(
        _mlp_body,
        grid=grid,
        in_specs=[
            pl.BlockSpec((_ROWS, NUM_CONT), lambda i: (i, 0)),
            pl.BlockSpec((_ROWS, NUM_CAT * EMB), lambda i: (i, 0)),
            full(NUM_CONT, H_CONT),
            full(H_CONT),
            full(H_CONT),
            full(H_CONT),
            full(H_CONT, H1),
            full(NUM_CAT * EMB, H1),
            full(H1),
            full(H1, H2),
            full(H2),
        ],
        out_specs=pl.BlockSpec((_ROWS, H2), lambda i: (i, 0)),
        out_shape=jax.ShapeDtypeStruct((B, H2), jnp.float32),
    )(x_cont, cat, w1t, b1, ln_g, ln_b, w2ht, w2ct, b2, w3t, b3)


def kernel(x_cont, x_cat, tables, W1, b1, ln_g, ln_b, W2, b2, W3, b3):
    offs = (jnp.arange(NUM_CAT, dtype=jnp.int32) * VOCAB)[None, :]
    idx = (x_cat + offs).reshape(NW, NG, G)
    table_flat = tables.reshape(NUM_CAT * VOCAB, EMB)
    cat = _sc_gather(idx, table_flat).reshape(B, NUM_CAT * EMB)
    return _mlp_call(
        x_cont, cat,
        W1.T, b1, ln_g, ln_b,
        W2[:, :H_CONT].T, W2[:, H_CONT:].T, b2,
        W3.T, b3,
    )


# speedup vs baseline: 28.2013x; 28.2013x over previous
import functools

import jax
import jax.numpy as jnp
from jax import lax
from jax.experimental import pallas as pl
from jax.experimental.pallas import tpu as pltpu
from jax.experimental.pallas import tpu_sc as plsc

B = 16384
NUM_CONT = 13
NUM_CAT = 26
VOCAB = 100001
EMB = 32
H_CONT = 32
H1 = 64
H2 = 32

NROWS = NUM_CAT * EMB
NW = 32
PAIRS_PER_W = NROWS // NW
CHUNK = 2048
NCHUNK = B // CHUNK
UNROLL = 8


def _make_sc_gather():
    mesh = plsc.VectorSubcoreMesh(core_axis_name="c", subcore_axis_name="s")

    @functools.partial(
        pl.kernel,
        mesh=mesh,
        out_type=jax.ShapeDtypeStruct((NROWS, B), jnp.float32),
        scratch_types=[
            pltpu.VMEM((VOCAB,), jnp.float32),
            pltpu.VMEM((CHUNK,), jnp.int32),
            pltpu.VMEM((CHUNK,), jnp.float32),
        ],
        compiler_params=pltpu.CompilerParams(needs_layout_passes=False),
    )
    def gather_k(idx_hbm, tab_hbm, out_hbm, row_v, idx_v, out_v):
        wid = lax.axis_index("s") * 2 + lax.axis_index("c")

        def pair_body(k, carry):
            p = wid * PAIRS_PER_W + k
            f = lax.shift_right_logical(p, 5)
            pltpu.sync_copy(tab_hbm.at[p], row_v)

            def chunk_body(c, carry2):
                pltpu.sync_copy(idx_hbm.at[f, pl.ds(c * CHUNK, CHUNK)], idx_v)

                def grp(j, carry3):
                    for u in range(UNROLL):
                        base = (j * UNROLL + u) * 16
                        iv = idx_v[pl.ds(base, 16)]
                        out_v[pl.ds(base, 16)] = plsc.load_gather(row_v, [iv])
                    return carry3

                lax.fori_loop(0, CHUNK // (16 * UNROLL), grp, 0)
                pltpu.sync_copy(out_v, out_hbm.at[p, pl.ds(c * CHUNK, CHUNK)])
                return carry2

            lax.fori_loop(0, NCHUNK, chunk_body, 0)
            return carry

        lax.fori_loop(0, PAIRS_PER_W, pair_body, 0)

    return gather_k


_sc_gather = _make_sc_gather()


def _mlp_body(xct_ref, cat_ref, w1_ref, b1_ref, g_ref, be_ref,
              w2h_ref, w2c_ref, b2_ref, w3_ref, b3_ref, out_ref):
    h = jnp.dot(w1_ref[...], xct_ref[...],
                preferred_element_type=jnp.float32) + b1_ref[...]
    mu = jnp.mean(h, axis=0, keepdims=True)
    var = jnp.mean((h - mu) * (h - mu), axis=0, keepdims=True)
    h = (h - mu) * lax.rsqrt(var + 1e-5) * g_ref[...] + be_ref[...]
    h = h * (1.0 / (1.0 + jnp.exp(-h)))
    z = (jnp.dot(w2h_ref[...], h, preferred_element_type=jnp.float32)
         + jnp.dot(w2c_ref[...], cat_ref[...],
                   preferred_element_type=jnp.float32)
         + b2_ref[...])
    z = z * (1.0 / (1.0 + jnp.exp(-z)))
    z = jnp.dot(w3_ref[...], z, preferred_element_type=jnp.float32) + b3_ref[...]
    out_ref[...] = z * (1.0 / (1.0 + jnp.exp(-z)))


_BCOL = 2048


def _mlp_call(xct, catt, w1, b1, ln_g, ln_b, w2h, w2c, b2, w3, b3):
    grid = (B // _BCOL,)
    full = lambda *shape: pl.BlockSpec(shape, lambda i: (0,) * len(shape))
    return pl.pallas_call(
        _mlp_body,
        grid=grid,
        in_specs=[
            pl.BlockSpec((NUM_CONT, _BCOL), lambda i: (0, i)),
            pl.BlockSpec((NROWS, _BCOL), lambda i: (0, i)),
            full(H_CONT, NUM_CONT),
            full(H_CONT, 1),
            full(H_CONT, 1),
            full(H_CONT, 1),
            full(H1, H_CONT),
            full(H1, NROWS),
            full(H1, 1),
            full(H2, H1),
            full(H2, 1),
        ],
        out_specs=pl.BlockSpec((H2, _BCOL), lambda i: (0, i)),
        out_shape=jax.ShapeDtypeStruct((H2, B), jnp.float32),
    )(xct, catt, w1, b1, ln_g, ln_b, w2h, w2c, b2, w3, b3)


def kernel(x_cont, x_cat, tables, W1, b1, ln_g, ln_b, W2, b2, W3, b3):
    tab_t = jnp.transpose(tables, (0, 2, 1)).reshape(NROWS, VOCAB)
    idx_t = x_cat.T
    cat_t = _sc_gather(idx_t, tab_t)
    out_t = _mlp_call(
        x_cont.T, cat_t,
        W1, b1.reshape(-1, 1), ln_g.reshape(-1, 1), ln_b.reshape(-1, 1),
        W2[:, :H_CONT], W2[:, H_CONT:], b2.reshape(-1, 1),
        W3, b3.reshape(-1, 1),
    )
    return out_t.T


# speedup vs baseline: 28.6743x; 1.0168x over previous
import functools

import jax
import jax.numpy as jnp
from jax import lax
from jax.experimental import pallas as pl
from jax.experimental.pallas import tpu as pltpu
from jax.experimental.pallas import tpu_sc as plsc

B = 16384
NUM_CONT = 13
NUM_CAT = 26
VOCAB = 100001
EMB = 32
H_CONT = 32
H1 = 64
H2 = 32

NROWS = NUM_CAT * EMB
NW = 32
PAIRS_PER_W = NROWS // NW
LO = 50048
HI = VOCAB - LO
CHUNK = 8192
UNROLL = 16
GRP_ITERS = CHUNK // (16 * UNROLL)


def _make_sc_gather():
    mesh = plsc.VectorSubcoreMesh(core_axis_name="c", subcore_axis_name="s")

    @functools.partial(
        pl.kernel,
        mesh=mesh,
        out_type=jax.ShapeDtypeStruct((NROWS, B), jnp.float32),
        scratch_types=[
            pltpu.VMEM((LO,), jnp.float32),
            pltpu.VMEM((HI,), jnp.float32),
            pltpu.VMEM((CHUNK,), jnp.int32),
            pltpu.VMEM((B,), jnp.float32),
            pltpu.SemaphoreType.DMA,
            pltpu.SemaphoreType.DMA,
        ],
        compiler_params=pltpu.CompilerParams(needs_layout_passes=False),
    )
    def gather_k(idx_hbm, tab_hbm, out_hbm, lo_v, hi_v, idx_v, out_v,
                 sem_hi, sem_lo):
        wid = lax.axis_index("s") * 2 + lax.axis_index("c")
        p0 = wid * PAIRS_PER_W
        pltpu.sync_copy(tab_hbm.at[p0, pl.ds(0, LO)], lo_v)

        def pass_lo(bc):
            def grp(j, carry3):
                for u in range(UNROLL):
                    base = (j * UNROLL + u) * 16
                    iv = idx_v[pl.ds(base, 16)]
                    ivl = jnp.minimum(iv, LO - 1)
                    out_v[pl.ds(bc * CHUNK + base, 16)] = (
                        plsc.load_gather(lo_v, [ivl]))
                return carry3

            lax.fori_loop(0, GRP_ITERS, grp, 0)

        def pass_hi(bc):
            def grp(j, carry3):
                for u in range(UNROLL):
                    base = (j * UNROLL + u) * 16
                    iv = idx_v[pl.ds(base, 16)]
                    ivh = jnp.clip(iv - LO, 0, HI - 1)
                    g = plsc.load_gather(hi_v, [ivh])
                    cur = out_v[pl.ds(bc * CHUNK + base, 16)]
                    out_v[pl.ds(bc * CHUNK + base, 16)] = (
                        jnp.where(iv >= LO, g, cur))
                return carry3

            lax.fori_loop(0, GRP_ITERS, grp, 0)

        def pair_body(k, carry):
            p = p0 + k
            f = lax.shift_right_logical(p, 5)
            hi_dma = pltpu.make_async_copy(
                tab_hbm.at[p, pl.ds(LO, HI)], hi_v, sem_hi)
            hi_dma.start()
            pltpu.sync_copy(idx_hbm.at[f, pl.ds(0, CHUNK)], idx_v)
            pass_lo(0)
            pltpu.sync_copy(idx_hbm.at[f, pl.ds(CHUNK, CHUNK)], idx_v)
            pass_lo(1)
            hi_dma.wait()
            pn = jnp.minimum(p + 1, NROWS - 1)
            lo_dma = pltpu.make_async_copy(
                tab_hbm.at[pn, pl.ds(0, LO)], lo_v, sem_lo)
            lo_dma.start()
            pass_hi(1)
            pltpu.sync_copy(idx_hbm.at[f, pl.ds(0, CHUNK)], idx_v)
            pass_hi(0)
            pltpu.sync_copy(out_v, out_hbm.at[p])
            lo_dma.wait()
            return carry

        lax.fori_loop(0, PAIRS_PER_W, pair_body, 0)

    return gather_k


_sc_gather = _make_sc_gather()


def _mlp_body(xct_ref, cat_ref, w1_ref, b1_ref, g_ref, be_ref,
              w2h_ref, w2c_ref, b2_ref, w3_ref, b3_ref, out_ref):
    h = jnp.dot(w1_ref[...], xct_ref[...],
                preferred_element_type=jnp.float32) + b1_ref[...]
    mu = jnp.mean(h, axis=0, keepdims=True)
    var = jnp.mean((h - mu) * (h - mu), axis=0, keepdims=True)
    h = (h - mu) * lax.rsqrt(var + 1e-5) * g_ref[...] + be_ref[...]
    h = h * (1.0 / (1.0 + jnp.exp(-h)))
    z = (jnp.dot(w2h_ref[...], h, preferred_element_type=jnp.float32)
         + jnp.dot(w2c_ref[...], cat_ref[...],
                   preferred_element_type=jnp.float32)
         + b2_ref[...])
    z = z * (1.0 / (1.0 + jnp.exp(-z)))
    z = jnp.dot(w3_ref[...], z, preferred_element_type=jnp.float32) + b3_ref[...]
    out_ref[...] = z * (1.0 / (1.0 + jnp.exp(-z)))


_BCOL = 2048


def _mlp_call(xct, catt, w1, b1, ln_g, ln_b, w2h, w2c, b2, w3, b3):
    grid = (B // _BCOL,)
    full = lambda *shape: pl.BlockSpec(shape, lambda i: (0,) * len(shape))
    return pl.pallas_call(
        _mlp_body,
        grid=grid,
        in_specs=[
            pl.BlockSpec((NUM_CONT, _BCOL), lambda i: (0, i)),
            pl.BlockSpec((NROWS, _BCOL), lambda i: (0, i)),
            full(H_CONT, NUM_CONT),
            full(H_CONT, 1),
            full(H_CONT, 1),
            full(H_CONT, 1),
            full(H1, H_CONT),
            full(H1, NROWS),
            full(H1, 1),
            full(H2, H1),
            full(H2, 1),
        ],
        out_specs=pl.BlockSpec((H2, _BCOL), lambda i: (0, i)),
        out_shape=jax.ShapeDtypeStruct((H2, B), jnp.float32),
    )(xct, catt, w1, b1, ln_g, ln_b, w2h, w2c, b2, w3, b3)


def kernel(x_cont, x_cat, tables, W1, b1, ln_g, ln_b, W2, b2, W3, b3):
    tab_t = jnp.transpose(tables, (0, 2, 1)).reshape(NROWS, VOCAB)
    idx_t = x_cat.T
    cat_t = _sc_gather(idx_t, tab_t)
    out_t = _mlp_call(
        x_cont.T, cat_t,
        W1, b1.reshape(-1, 1), ln_g.reshape(-1, 1), ln_b.reshape(-1, 1),
        W2[:, :H_CONT], W2[:, H_CONT:], b2.reshape(-1, 1),
        W3, b3.reshape(-1, 1),
    )
    return out_t.T


# speedup vs baseline: 41.7312x; 1.4554x over previous
import functools

import jax
import jax.numpy as jnp
from jax import lax
from jax.experimental import pallas as pl
from jax.experimental.pallas import tpu as pltpu
from jax.experimental.pallas import tpu_sc as plsc

B = 16384
NUM_CONT = 13
NUM_CAT = 26
VOCAB = 100001
EMB = 32
H_CONT = 32
H1 = 64
H2 = 32

NROWS = NUM_CAT * EMB
NW = 32
PAIRS_PER_W = NROWS // NW
LO = 50048
HI = VOCAB - LO
CHUNK = 8192
UNROLL = 16
GRP_ITERS = CHUNK // (16 * UNROLL)


def _make_sc_gather():
    mesh = plsc.VectorSubcoreMesh(core_axis_name="c", subcore_axis_name="s")

    @functools.partial(
        pl.kernel,
        mesh=mesh,
        out_type=jax.ShapeDtypeStruct((NROWS, B), jnp.float32),
        scratch_types=[
            pltpu.VMEM((LO,), jnp.float32),
            pltpu.VMEM((HI,), jnp.float32),
            pltpu.VMEM((CHUNK,), jnp.int32),
            pltpu.VMEM((B,), jnp.float32),
            pltpu.SemaphoreType.DMA,
            pltpu.SemaphoreType.DMA,
        ],
        compiler_params=pltpu.CompilerParams(needs_layout_passes=False),
    )
    def gather_k(idx_hbm, tab_hbm, out_hbm, lo_v, hi_v, idx_v, out_v,
                 sem_hi, sem_lo):
        wid = lax.axis_index("s") * 2 + lax.axis_index("c")
        p0 = wid * PAIRS_PER_W
        pltpu.sync_copy(tab_hbm.at[p0, pl.ds(0, LO)], lo_v)

        def pass_lo(bc):
            @plsc.parallel_loop(0, CHUNK // 16, unroll=UNROLL)
            def _(g):
                base = g * 16
                iv = idx_v[pl.ds(base, 16)]
                ivl = jnp.minimum(iv, LO - 1)
                out_v[pl.ds(bc * CHUNK + base, 16)] = (
                    plsc.load_gather(lo_v, [ivl]))

        def pass_hi(bc):
            @plsc.parallel_loop(0, CHUNK // 16, unroll=UNROLL)
            def _(g):
                base = g * 16
                iv = idx_v[pl.ds(base, 16)]
                ivh = jnp.clip(iv - LO, 0, HI - 1)
                gth = plsc.load_gather(hi_v, [ivh])
                cur = out_v[pl.ds(bc * CHUNK + base, 16)]
                out_v[pl.ds(bc * CHUNK + base, 16)] = (
                    jnp.where(iv >= LO, gth, cur))

        def pair_body(k, carry):
            p = p0 + k
            f = lax.shift_right_logical(p, 5)
            hi_dma = pltpu.make_async_copy(
                tab_hbm.at[p, pl.ds(LO, HI)], hi_v, sem_hi)
            hi_dma.start()
            pltpu.sync_copy(idx_hbm.at[f, pl.ds(0, CHUNK)], idx_v)
            pass_lo(0)
            pltpu.sync_copy(idx_hbm.at[f, pl.ds(CHUNK, CHUNK)], idx_v)
            pass_lo(1)
            hi_dma.wait()
            pn = jnp.minimum(p + 1, NROWS - 1)
            lo_dma = pltpu.make_async_copy(
                tab_hbm.at[pn, pl.ds(0, LO)], lo_v, sem_lo)
            lo_dma.start()
            pass_hi(1)
            pltpu.sync_copy(idx_hbm.at[f, pl.ds(0, CHUNK)], idx_v)
            pass_hi(0)
            pltpu.sync_copy(out_v, out_hbm.at[p])
            lo_dma.wait()
            return carry

        lax.fori_loop(0, PAIRS_PER_W, pair_body, 0)

    return gather_k


_sc_gather = _make_sc_gather()


def _mlp_body(xct_ref, cat_ref, w1_ref, b1_ref, g_ref, be_ref,
              w2h_ref, w2c_ref, b2_ref, w3_ref, b3_ref, out_ref):
    h = jnp.dot(w1_ref[...], xct_ref[...],
                preferred_element_type=jnp.float32) + b1_ref[...]
    mu = jnp.mean(h, axis=0, keepdims=True)
    var = jnp.mean((h - mu) * (h - mu), axis=0, keepdims=True)
    h = (h - mu) * lax.rsqrt(var + 1e-5) * g_ref[...] + be_ref[...]
    h = h * (1.0 / (1.0 + jnp.exp(-h)))
    z = (jnp.dot(w2h_ref[...], h, preferred_element_type=jnp.float32)
         + jnp.dot(w2c_ref[...], cat_ref[...],
                   preferred_element_type=jnp.float32)
         + b2_ref[...])
    z = z * (1.0 / (1.0 + jnp.exp(-z)))
    z = jnp.dot(w3_ref[...], z, preferred_element_type=jnp.float32) + b3_ref[...]
    out_ref[...] = z * (1.0 / (1.0 + jnp.exp(-z)))


_BCOL = 2048


def _mlp_call(xct, catt, w1, b1, ln_g, ln_b, w2h, w2c, b2, w3, b3):
    grid = (B // _BCOL,)
    full = lambda *shape: pl.BlockSpec(shape, lambda i: (0,) * len(shape))
    return pl.pallas_call(
        _mlp_body,
        grid=grid,
        in_specs=[
            pl.BlockSpec((NUM_CONT, _BCOL), lambda i: (0, i)),
            pl.BlockSpec((NROWS, _BCOL), lambda i: (0, i)),
            full(H_CONT, NUM_CONT),
            full(H_CONT, 1),
            full(H_CONT, 1),
            full(H_CONT, 1),
            full(H1, H_CONT),
            full(H1, NROWS),
            full(H1, 1),
            full(H2, H1),
            full(H2, 1),
        ],
        out_specs=pl.BlockSpec((H2, _BCOL), lambda i: (0, i)),
        out_shape=jax.ShapeDtypeStruct((H2, B), jnp.float32),
    )(xct, catt, w1, b1, ln_g, ln_b, w2h, w2c, b2, w3, b3)


def kernel(x_cont, x_cat, tables, W1, b1, ln_g, ln_b, W2, b2, W3, b3):
    tab_t = jnp.transpose(tables, (0, 2, 1)).reshape(NROWS, VOCAB)
    idx_t = x_cat.T
    cat_t = _sc_gather(idx_t, tab_t)
    out_t = _mlp_call(
        x_cont.T, cat_t,
        W1, b1.reshape(-1, 1), ln_g.reshape(-1, 1), ln_b.reshape(-1, 1),
        W2[:, :H_CONT], W2[:, H_CONT:], b2.reshape(-1, 1),
        W3, b3.reshape(-1, 1),
    )
    return out_t.T


# speedup vs baseline: 43.3513x; 1.0388x over previous
import functools

import jax
import jax.numpy as jnp
from jax import lax
from jax.experimental import pallas as pl
from jax.experimental.pallas import tpu as pltpu
from jax.experimental.pallas import tpu_sc as plsc

B = 16384
NUM_CONT = 13
NUM_CAT = 26
VOCAB = 100001
EMB = 32
H_CONT = 32
H1 = 64
H2 = 32

NROWS = NUM_CAT * EMB
NW = 32
PAIRS_PER_W = NROWS // NW
SEG = 33280
TAIL_OFF = 3 * SEG
TAIL_LEN = VOCAB - TAIL_OFF
UNROLL = 16
NGRP = B // 16


def _make_sc_gather():
    mesh = plsc.VectorSubcoreMesh(core_axis_name="c", subcore_axis_name="s")

    @functools.partial(
        pl.kernel,
        mesh=mesh,
        out_type=jax.ShapeDtypeStruct((NROWS, B), jnp.float32),
        scratch_types=[
            pltpu.VMEM((SEG,), jnp.float32),
            pltpu.VMEM((SEG,), jnp.float32),
            pltpu.VMEM((TAIL_LEN,), jnp.float32),
            pltpu.VMEM((B,), jnp.int32),
            pltpu.VMEM((B,), jnp.float32),
            pltpu.SemaphoreType.DMA,
            pltpu.SemaphoreType.DMA,
            pltpu.SemaphoreType.DMA,
            pltpu.SemaphoreType.DMA,
        ],
        compiler_params=pltpu.CompilerParams(needs_layout_passes=False),
    )
    def gather_k(idx_hbm, tab_hbm, out_hbm, buf_a, buf_b, tail_v, idx_v,
                 out_v, sem_a, sem_b, sem_t, sem_o):
        wid = lax.axis_index("s") * 2 + lax.axis_index("c")
        p0 = wid * PAIRS_PER_W
        bufs = (buf_a, buf_b)
        sems = (sem_a, sem_b)
        lanes = lax.iota(jnp.int32, 16)

        def start_seg(p, t, b):
            return pltpu.make_async_copy(
                tab_hbm.at[p, pl.ds(t * SEG, SEG)], bufs[b], sems[b])

        def pass_t(t, b):
            buf = bufs[b]
            if t == 0:
                @plsc.parallel_loop(0, NGRP, unroll=UNROLL)
                def _(g):
                    base = g * 16
                    iv = idx_v[pl.ds(base, 16)]
                    out_v[pl.ds(base, 16)] = plsc.load_gather(
                        buf, [jnp.minimum(iv, SEG - 1)])
            else:
                off = t * SEG

                @plsc.parallel_loop(0, NGRP, unroll=UNROLL)
                def _(g):
                    base = g * 16
                    iv = idx_v[pl.ds(base, 16)]
                    ivl = jnp.clip(iv - off, 0, SEG - 1)
                    gth = plsc.load_gather(buf, [ivl])
                    plsc.store_scatter(out_v, [base + lanes], gth,
                                       mask=iv >= off)

        def pass_tail():
            @plsc.parallel_loop(0, NGRP, unroll=UNROLL)
            def _(g):
                base = g * 16
                iv = idx_v[pl.ds(base, 16)]
                ivl = jnp.clip(iv - TAIL_OFF, 0, TAIL_LEN - 1)
                gth = plsc.load_gather(tail_v, [ivl])
                plsc.store_scatter(out_v, [base + lanes], gth,
                                   mask=iv >= TAIL_OFF)

        pltpu.sync_copy(idx_hbm.at[lax.shift_right_logical(p0, 5)], idx_v)
        pltpu.sync_copy(tab_hbm.at[p0, pl.ds(0, SEG)], buf_a)

        def pair2_body(j, carry):
            for i in range(2):
                k = j * 2 + i
                p = p0 + k
                f = lax.shift_right_logical(p, 5)
                @pl.when(k > 0)
                def _():
                    pltpu.make_async_copy(out_v, out_hbm.at[p], sem_o).wait()

                @pl.when(jnp.logical_and(jnp.bitwise_and(p, 31) == 0, k > 0))
                def _():
                    pltpu.sync_copy(idx_hbm.at[f], idx_v)

                dt = pltpu.make_async_copy(
                    tab_hbm.at[p, pl.ds(TAIL_OFF, TAIL_LEN)], tail_v, sem_t)
                dt.start()
                r = i
                d1 = start_seg(p, 1, 1 - r)
                d1.start()
                pass_t(0, r)
                d1.wait()
                d2 = start_seg(p, 2, r)
                d2.start()
                pass_t(1, 1 - r)
                d2.wait()
                pn = jnp.minimum(p + 1, NROWS - 1)
                d0 = start_seg(pn, 0, 1 - r)
                d0.start()
                pass_t(2, r)
                dt.wait()
                pass_tail()
                pltpu.make_async_copy(out_v, out_hbm.at[p], sem_o).start()
                d0.wait()
            return carry

        lax.fori_loop(0, PAIRS_PER_W // 2, pair2_body, 0)
        pltpu.make_async_copy(out_v, out_hbm.at[p0], sem_o).wait()

    return gather_k


_sc_gather = _make_sc_gather()


def _mlp_body(xct_ref, cat_ref, w1_ref, b1_ref, g_ref, be_ref,
              w2h_ref, w2c_ref, b2_ref, w3_ref, b3_ref, out_ref):
    h = jnp.dot(w1_ref[...], xct_ref[...],
                preferred_element_type=jnp.float32) + b1_ref[...]
    mu = jnp.mean(h, axis=0, keepdims=True)
    var = jnp.mean((h - mu) * (h - mu), axis=0, keepdims=True)
    h = (h - mu) * lax.rsqrt(var + 1e-5) * g_ref[...] + be_ref[...]
    h = h * (1.0 / (1.0 + jnp.exp(-h)))
    z = (jnp.dot(w2h_ref[...], h, preferred_element_type=jnp.float32)
         + jnp.dot(w2c_ref[...], cat_ref[...],
                   preferred_element_type=jnp.float32)
         + b2_ref[...])
    z = z * (1.0 / (1.0 + jnp.exp(-z)))
    z = jnp.dot(w3_ref[...], z, preferred_element_type=jnp.float32) + b3_ref[...]
    out_ref[...] = z * (1.0 / (1.0 + jnp.exp(-z)))


_BCOL = 2048


def _mlp_call(xct, catt, w1, b1, ln_g, ln_b, w2h, w2c, b2, w3, b3):
    grid = (B // _BCOL,)
    full = lambda *shape: pl.BlockSpec(shape, lambda i: (0,) * len(shape))
    return pl.pallas_call(
        _mlp_body,
        grid=grid,
        in_specs=[
            pl.BlockSpec((NUM_CONT, _BCOL), lambda i: (0, i)),
            pl.BlockSpec((NROWS, _BCOL), lambda i: (0, i)),
            full(H_CONT, NUM_CONT),
            full(H_CONT, 1),
            full(H_CONT, 1),
            full(H_CONT, 1),
            full(H1, H_CONT),
            full(H1, NROWS),
            full(H1, 1),
            full(H2, H1),
            full(H2, 1),
        ],
        out_specs=pl.BlockSpec((H2, _BCOL), lambda i: (0, i)),
        out_shape=jax.ShapeDtypeStruct((H2, B), jnp.float32),
    )(xct, catt, w1, b1, ln_g, ln_b, w2h, w2c, b2, w3, b3)


def kernel(x_cont, x_cat, tables, W1, b1, ln_g, ln_b, W2, b2, W3, b3):
    tab_t = jnp.transpose(tables, (0, 2, 1)).reshape(NROWS, VOCAB)
    idx_t = x_cat.T
    cat_t = _sc_gather(idx_t, tab_t)
    out_t = _mlp_call(
        x_cont.T, cat_t,
        W1, b1.reshape(-1, 1), ln_g.reshape(-1, 1), ln_b.reshape(-1, 1),
        W2[:, :H_CONT], W2[:, H_CONT:], b2.reshape(-1, 1),
        W3, b3.reshape(-1, 1),
    )
    return out_t.T


# speedup vs baseline: 45.4220x; 1.0478x over previous
import functools

import jax
import jax.numpy as jnp
from jax import lax
from jax.experimental import pallas as pl
from jax.experimental.pallas import tpu as pltpu
from jax.experimental.pallas import tpu_sc as plsc

B = 16384
NUM_CONT = 13
NUM_CAT = 26
VOCAB = 100001
EMB = 32
H_CONT = 32
H1 = 64
H2 = 32

NROWS = NUM_CAT * EMB
NW = 32
PAIRS_PER_W = NROWS // NW
SEG = 33280
TAIL_OFF = 3 * SEG
TAIL_LEN = VOCAB - TAIL_OFF
UNROLL = 16
NGRP = B // 16


def _make_sc_gather():
    mesh = plsc.VectorSubcoreMesh(core_axis_name="c", subcore_axis_name="s")

    @functools.partial(
        pl.kernel,
        mesh=mesh,
        out_type=jax.ShapeDtypeStruct((NROWS, B), jnp.float32),
        scratch_types=[
            pltpu.VMEM((SEG,), jnp.float32),
            pltpu.VMEM((SEG,), jnp.float32),
            pltpu.VMEM((TAIL_LEN,), jnp.float32),
            pltpu.VMEM((B,), jnp.int32),
            pltpu.VMEM((B,), jnp.float32),
            pltpu.VMEM((B,), jnp.float32),
            pltpu.SemaphoreType.DMA,
            pltpu.SemaphoreType.DMA,
            pltpu.SemaphoreType.DMA,
            pltpu.SemaphoreType.DMA,
            pltpu.SemaphoreType.DMA,
        ],
        compiler_params=pltpu.CompilerParams(needs_layout_passes=False),
    )
    def gather_k(idx_hbm, tab_hbm, out_hbm, buf_a, buf_b, tail_v, idx_v,
                 out_a, out_b, sem_a, sem_b, sem_t, sem_oa, sem_ob):
        wid = lax.axis_index("s") * 2 + lax.axis_index("c")
        p0 = wid * PAIRS_PER_W
        bufs = (buf_a, buf_b)
        sems = (sem_a, sem_b)
        outs = (out_a, out_b)
        osems = (sem_oa, sem_ob)
        lanes = lax.iota(jnp.int32, 16)

        def start_seg(p, t, b):
            return pltpu.make_async_copy(
                tab_hbm.at[p, pl.ds(t * SEG, SEG)], bufs[b], sems[b])

        def tail_copy(p):
            return pltpu.make_async_copy(
                tab_hbm.at[p, pl.ds(TAIL_OFF, TAIL_LEN)], tail_v, sem_t)

        def out_copy(p, q):
            return pltpu.make_async_copy(outs[q], out_hbm.at[p], osems[q])

        def pass_t(t, b, q):
            buf = bufs[b]
            out_v = outs[q]
            if t == 0:
                @plsc.parallel_loop(0, NGRP, unroll=UNROLL)
                def _(g):
                    base = g * 16
                    iv = idx_v[pl.ds(base, 16)]
                    out_v[pl.ds(base, 16)] = plsc.load_gather(
                        buf, [jnp.minimum(iv, SEG - 1)])
            else:
                off = t * SEG

                @plsc.parallel_loop(0, NGRP, unroll=UNROLL)
                def _(g):
                    base = g * 16
                    iv = idx_v[pl.ds(base, 16)]
                    ivl = jnp.clip(iv - off, 0, SEG - 1)
                    gth = plsc.load_gather(buf, [ivl])
                    plsc.store_scatter(out_v, [base + lanes], gth,
                                       mask=iv >= off)

        def pass_tail(q):
            out_v = outs[q]

            @plsc.parallel_loop(0, NGRP, unroll=UNROLL)
            def _(g):
                base = g * 16
                iv = idx_v[pl.ds(base, 16)]
                ivl = jnp.clip(iv - TAIL_OFF, 0, TAIL_LEN - 1)
                gth = plsc.load_gather(tail_v, [ivl])
                plsc.store_scatter(out_v, [base + lanes], gth,
                                   mask=iv >= TAIL_OFF)

        pltpu.sync_copy(idx_hbm.at[lax.shift_right_logical(p0, 5)], idx_v)
        pltpu.sync_copy(tab_hbm.at[p0, pl.ds(0, SEG)], buf_a)

        def pair2_body(j, carry):
            for i in range(2):
                k = j * 2 + i
                p = p0 + k
                f = lax.shift_right_logical(p, 5)
                r = i
                q = i
                d1 = start_seg(p, 1, 1 - r)
                d1.start()
                @pl.when(k > 0)
                def _():
                    tail_copy(p).wait()
                    pass_tail(1 - q)
                    out_copy(p - 1, 1 - q).start()

                @pl.when(jnp.logical_and(jnp.bitwise_and(p, 31) == 0, k > 0))
                def _():
                    pltpu.sync_copy(idx_hbm.at[f], idx_v)

                dt = tail_copy(p)
                dt.start()
                @pl.when(k >= 2)
                def _():
                    out_copy(p, q).wait()

                pass_t(0, r, q)
                d1.wait()
                d2 = start_seg(p, 2, r)
                d2.start()
                pass_t(1, 1 - r, q)
                d2.wait()
                pn = jnp.minimum(p + 1, NROWS - 1)
                d0 = start_seg(pn, 0, 1 - r)
                d0.start()
                pass_t(2, r, q)
                d0.wait()
            return carry

        lax.fori_loop(0, PAIRS_PER_W // 2, pair2_body, 0)
        p_last = p0 + PAIRS_PER_W - 1
        tail_copy(p_last).wait()
        pass_tail(1)
        out_copy(p_last, 1).start()
        out_copy(p0, 0).wait()
        out_copy(p0, 1).wait()

    return gather_k


_sc_gather = _make_sc_gather()


def _mlp_body(xct_ref, cat_ref, w1_ref, b1_ref, g_ref, be_ref,
              w2h_ref, w2c_ref, b2_ref, w3_ref, b3_ref, out_ref):
    h = jnp.dot(w1_ref[...], xct_ref[...],
                preferred_element_type=jnp.float32) + b1_ref[...]
    mu = jnp.mean(h, axis=0, keepdims=True)
    var = jnp.mean((h - mu) * (h - mu), axis=0, keepdims=True)
    h = (h - mu) * lax.rsqrt(var + 1e-5) * g_ref[...] + be_ref[...]
    h = h * (1.0 / (1.0 + jnp.exp(-h)))
    z = (jnp.dot(w2h_ref[...], h, preferred_element_type=jnp.float32)
         + jnp.dot(w2c_ref[...], cat_ref[...],
                   preferred_element_type=jnp.float32)
         + b2_ref[...])
    z = z * (1.0 / (1.0 + jnp.exp(-z)))
    z = jnp.dot(w3_ref[...], z, preferred_element_type=jnp.float32) + b3_ref[...]
    out_ref[...] = z * (1.0 / (1.0 + jnp.exp(-z)))


_BCOL = 2048


def _mlp_call(xct, catt, w1, b1, ln_g, ln_b, w2h, w2c, b2, w3, b3):
    grid = (B // _BCOL,)
    full = lambda *shape: pl.BlockSpec(shape, lambda i: (0,) * len(shape))
    return pl.pallas_call(
        _mlp_body,
        grid=grid,
        in_specs=[
            pl.BlockSpec((NUM_CONT, _BCOL), lambda i: (0, i)),
            pl.BlockSpec((NROWS, _BCOL), lambda i: (0, i)),
            full(H_CONT, NUM_CONT),
            full(H_CONT, 1),
            full(H_CONT, 1),
            full(H_CONT, 1),
            full(H1, H_CONT),
            full(H1, NROWS),
            full(H1, 1),
            full(H2, H1),
            full(H2, 1),
        ],
        out_specs=pl.BlockSpec((H2, _BCOL), lambda i: (0, i)),
        out_shape=jax.ShapeDtypeStruct((H2, B), jnp.float32),
    )(xct, catt, w1, b1, ln_g, ln_b, w2h, w2c, b2, w3, b3)


def kernel(x_cont, x_cat, tables, W1, b1, ln_g, ln_b, W2, b2, W3, b3):
    tab_t = jnp.transpose(tables, (0, 2, 1)).reshape(NROWS, VOCAB)
    idx_t = x_cat.T
    cat_t = _sc_gather(idx_t, tab_t)
    out_t = _mlp_call(
        x_cont.T, cat_t,
        W1, b1.reshape(-1, 1), ln_g.reshape(-1, 1), ln_b.reshape(-1, 1),
        W2[:, :H_CONT], W2[:, H_CONT:], b2.reshape(-1, 1),
        W3, b3.reshape(-1, 1),
    )
    return out_t.T


# speedup vs baseline: 45.8452x; 1.0093x over previous
import functools

import jax
import jax.numpy as jnp
from jax import lax
from jax.experimental import pallas as pl
from jax.experimental.pallas import tpu as pltpu
from jax.experimental.pallas import tpu_sc as plsc

B = 16384
NUM_CONT = 13
NUM_CAT = 26
VOCAB = 100001
EMB = 32
H_CONT = 32
H1 = 64
H2 = 32

NROWS = NUM_CAT * EMB
NW = 32
PAIRS_PER_W = NROWS // NW
SEG_LEN = (38272, 38528, 23040)
SEG_OFF = (0, 38272, 76800)
SEG_MAX = 38528
TAIL_OFF = 99840
TAIL_LEN = VOCAB - TAIL_OFF
UNROLL = 16
NGRP = B // 16


def _make_sc_gather():
    mesh = plsc.VectorSubcoreMesh(core_axis_name="c", subcore_axis_name="s")

    @functools.partial(
        pl.kernel,
        mesh=mesh,
        out_type=jax.ShapeDtypeStruct((NROWS, B), jnp.float32),
        scratch_types=[
            pltpu.VMEM((SEG_MAX,), jnp.float32),
            pltpu.VMEM((SEG_MAX,), jnp.float32),
            pltpu.VMEM((TAIL_LEN,), jnp.float32),
            pltpu.VMEM((B,), jnp.int32),
            pltpu.VMEM((B,), jnp.float32),
            pltpu.VMEM((B,), jnp.float32),
            pltpu.SemaphoreType.DMA,
            pltpu.SemaphoreType.DMA,
            pltpu.SemaphoreType.DMA,
            pltpu.SemaphoreType.DMA,
            pltpu.SemaphoreType.DMA,
        ],
        compiler_params=pltpu.CompilerParams(needs_layout_passes=False),
    )
    def gather_k(idx_hbm, tab_hbm, out_hbm, buf_a, buf_b, tail_v, idx_v,
                 out_a, out_b, sem_a, sem_b, sem_t, sem_oa, sem_ob):
        wid = lax.axis_index("s") * 2 + lax.axis_index("c")
        p0 = wid * PAIRS_PER_W
        bufs = (buf_a, buf_b)
        sems = (sem_a, sem_b)
        outs = (out_a, out_b)
        osems = (sem_oa, sem_ob)
        lanes = lax.iota(jnp.int32, 16)

        def start_seg(p, t, b):
            return pltpu.make_async_copy(
                tab_hbm.at[p, pl.ds(SEG_OFF[t], SEG_LEN[t])],
                bufs[b].at[pl.ds(0, SEG_LEN[t])], sems[b])

        def tail_copy(p):
            return pltpu.make_async_copy(
                tab_hbm.at[p, pl.ds(TAIL_OFF, TAIL_LEN)], tail_v, sem_t)

        def out_copy(p, q):
            return pltpu.make_async_copy(outs[q], out_hbm.at[p], osems[q])

        def pass_t(t, b, q):
            buf = bufs[b]
            out_v = outs[q]
            if t == 0:
                @plsc.parallel_loop(0, NGRP, unroll=UNROLL)
                def _(g):
                    base = g * 16
                    iv = idx_v[pl.ds(base, 16)]
                    out_v[pl.ds(base, 16)] = plsc.load_gather(
                        buf, [jnp.minimum(iv, SEG_LEN[0] - 1)])
            else:
                off = SEG_OFF[t]
                cap = SEG_LEN[t] - 1

                @plsc.parallel_loop(0, NGRP, unroll=UNROLL)
                def _(g):
                    base = g * 16
                    iv = idx_v[pl.ds(base, 16)]
                    ivl = jnp.clip(iv - off, 0, cap)
                    gth = plsc.load_gather(buf, [ivl])
                    plsc.store_scatter(out_v, [base + lanes], gth,
                                       mask=iv >= off)

        def pass_tail(q):
            out_v = outs[q]

            @plsc.parallel_loop(0, NGRP, unroll=UNROLL)
            def _(g):
                base = g * 16
                iv = idx_v[pl.ds(base, 16)]
                ivl = jnp.clip(iv - TAIL_OFF, 0, TAIL_LEN - 1)
                gth = plsc.load_gather(tail_v, [ivl])
                plsc.store_scatter(out_v, [base + lanes], gth,
                                   mask=iv >= TAIL_OFF)

        pltpu.sync_copy(idx_hbm.at[lax.shift_right_logical(p0, 5)], idx_v)
        pltpu.sync_copy(tab_hbm.at[p0, pl.ds(0, SEG_LEN[0])],
                        buf_a.at[pl.ds(0, SEG_LEN[0])])

        def pair2_body(j, carry):
            for i in range(2):
                k = j * 2 + i
                p = p0 + k
                f = lax.shift_right_logical(p, 5)
                r = i
                q = i
                d1 = start_seg(p, 1, 1 - r)
                d1.start()
                @pl.when(k > 0)
                def _():
                    tail_copy(p).wait()
                    pass_tail(1 - q)
                    out_copy(p - 1, 1 - q).start()

                @pl.when(jnp.logical_and(jnp.bitwise_and(p, 31) == 0, k > 0))
                def _():
                    pltpu.sync_copy(idx_hbm.at[f], idx_v)

                dt = tail_copy(p)
                dt.start()
                @pl.when(k >= 2)
                def _():
                    out_copy(p, q).wait()

                pass_t(0, r, q)
                d1.wait()
                d2 = start_seg(p, 2, r)
                d2.start()
                pass_t(1, 1 - r, q)
                d2.wait()
                pn = jnp.minimum(p + 1, NROWS - 1)
                d0 = start_seg(pn, 0, 1 - r)
                d0.start()
                pass_t(2, r, q)
                d0.wait()
            return carry

        lax.fori_loop(0, PAIRS_PER_W // 2, pair2_body, 0)
        p_last = p0 + PAIRS_PER_W - 1
        tail_copy(p_last).wait()
        pass_tail(1)
        out_copy(p_last, 1).start()
        out_copy(p0, 0).wait()
        out_copy(p0, 1).wait()

    return gather_k


_sc_gather = _make_sc_gather()


def _mlp_body(xct_ref, cat_ref, w1_ref, b1_ref, g_ref, be_ref,
              w2h_ref, w2c_ref, b2_ref, w3_ref, b3_ref, out_ref):
    h = jnp.dot(w1_ref[...], xct_ref[...],
                preferred_element_type=jnp.float32) + b1_ref[...]
    mu = jnp.mean(h, axis=0, keepdims=True)
    var = jnp.mean((h - mu) * (h - mu), axis=0, keepdims=True)
    h = (h - mu) * lax.rsqrt(var + 1e-5) * g_ref[...] + be_ref[...]
    h = h * (1.0 / (1.0 + jnp.exp(-h)))
    z = (jnp.dot(w2h_ref[...], h, preferred_element_type=jnp.float32)
         + jnp.dot(w2c_ref[...], cat_ref[...],
                   preferred_element_type=jnp.float32)
         + b2_ref[...])
    z = z * (1.0 / (1.0 + jnp.exp(-z)))
    z = jnp.dot(w3_ref[...], z, preferred_element_type=jnp.float32) + b3_ref[...]
    out_ref[...] = z * (1.0 / (1.0 + jnp.exp(-z)))


_BCOL = 2048


def _mlp_call(xct, catt, w1, b1, ln_g, ln_b, w2h, w2c, b2, w3, b3):
    grid = (B // _BCOL,)
    full = lambda *shape: pl.BlockSpec(shape, lambda i: (0,) * len(shape))
    return pl.pallas_call(
        _mlp_body,
        grid=grid,
        in_specs=[
            pl.BlockSpec((NUM_CONT, _BCOL), lambda i: (0, i)),
            pl.BlockSpec((NROWS, _BCOL), lambda i: (0, i)),
            full(H_CONT, NUM_CONT),
            full(H_CONT, 1),
            full(H_CONT, 1),
            full(H_CONT, 1),
            full(H1, H_CONT),
            full(H1, NROWS),
            full(H1, 1),
            full(H2, H1),
            full(H2, 1),
        ],
        out_specs=pl.BlockSpec((H2, _BCOL), lambda i: (0, i)),
        out_shape=jax.ShapeDtypeStruct((H2, B), jnp.float32),
    )(xct, catt, w1, b1, ln_g, ln_b, w2h, w2c, b2, w3, b3)


def kernel(x_cont, x_cat, tables, W1, b1, ln_g, ln_b, W2, b2, W3, b3):
    tab_t = jnp.transpose(tables, (0, 2, 1)).reshape(NROWS, VOCAB)
    idx_t = x_cat.T
    cat_t = _sc_gather(idx_t, tab_t)
    out_t = _mlp_call(
        x_cont.T, cat_t,
        W1, b1.reshape(-1, 1), ln_g.reshape(-1, 1), ln_b.reshape(-1, 1),
        W2[:, :H_CONT], W2[:, H_CONT:], b2.reshape(-1, 1),
        W3, b3.reshape(-1, 1),
    )
    return out_t.T


# speedup vs baseline: 52.8601x; 1.1530x over previous
import functools

import jax
import jax.numpy as jnp
from jax import lax
from jax.experimental import pallas as pl
from jax.experimental.pallas import tpu as pltpu
from jax.experimental.pallas import tpu_sc as plsc

B = 16384
NUM_CONT = 13
NUM_CAT = 26
VOCAB = 100001
EMB = 32
H_CONT = 32
H1 = 64
H2 = 32

NROWS = NUM_CAT * EMB
NW = 32
PAIRS_PER_W = NROWS // NW
SEG_LEN = (38272, 38528, 23040)
SEG_OFF = (0, 38272, 76800)
TAIL_OFF = 99840
TAIL_LEN = VOCAB - TAIL_OFF
TAIL_PAD = 256
SEG_MAX = 38528
UNROLL = 16
NGRP = B // 16


def _make_sc_gather():
    mesh = plsc.VectorSubcoreMesh(core_axis_name="c", subcore_axis_name="s")

    @functools.partial(
        pl.kernel,
        mesh=mesh,
        out_type=jax.ShapeDtypeStruct((NROWS, B), jnp.float32),
        scratch_types=[
            pltpu.VMEM((SEG_MAX,), jnp.float32),
            pltpu.VMEM((SEG_MAX,), jnp.float32),
            pltpu.VMEM((B,), jnp.int32),
            pltpu.VMEM((B,), jnp.float32),
            pltpu.VMEM((B,), jnp.float32),
            pltpu.SemaphoreType.DMA,
            pltpu.SemaphoreType.DMA,
            pltpu.SemaphoreType.DMA,
            pltpu.SemaphoreType.DMA,
        ],
        compiler_params=pltpu.CompilerParams(needs_layout_passes=False),
    )
    def gather_k(idx_hbm, tab_hbm, tail_hbm, out_hbm, buf_a, buf_b, idx_v,
                 out_a, out_b, sem_a, sem_b, sem_oa, sem_ob):
        wid = lax.axis_index("s") * 2 + lax.axis_index("c")
        p0 = wid * PAIRS_PER_W
        bufs = (buf_a, buf_b)
        sems = (sem_a, sem_b)
        outs = (out_a, out_b)
        osems = (sem_oa, sem_ob)
        lanes = lax.iota(jnp.int32, 16)

        def start_seg(p, t, b):
            return pltpu.make_async_copy(
                tab_hbm.at[p, pl.ds(SEG_OFF[t], SEG_LEN[t])],
                bufs[b].at[pl.ds(0, SEG_LEN[t])], sems[b])

        def tail_copy(p, b):
            return pltpu.make_async_copy(
                tail_hbm.at[p],
                bufs[b].at[pl.ds(SEG_LEN[2], TAIL_PAD)], sems[b])

        def out_copy(p, q):
            return pltpu.make_async_copy(outs[q], out_hbm.at[p], osems[q])

        def pass_t(t, b, q):
            buf = bufs[b]
            out_v = outs[q]
            if t == 0:
                @plsc.parallel_loop(0, NGRP, unroll=UNROLL)
                def _(g):
                    base = g * 16
                    iv = idx_v[pl.ds(base, 16)]
                    out_v[pl.ds(base, 16)] = plsc.load_gather(
                        buf, [jnp.minimum(iv, SEG_LEN[0] - 1)])
            else:
                off = SEG_OFF[t]
                cap = SEG_LEN[t] - 1 if t == 1 else SEG_LEN[2] + TAIL_LEN - 1

                @plsc.parallel_loop(0, NGRP, unroll=UNROLL)
                def _(g):
                    base = g * 16
                    iv = idx_v[pl.ds(base, 16)]
                    ivl = jnp.clip(iv - off, 0, cap)
                    gth = plsc.load_gather(buf, [ivl])
                    plsc.store_scatter(out_v, [base + lanes], gth,
                                       mask=iv >= off)

        pltpu.sync_copy(idx_hbm.at[lax.shift_right_logical(p0, 5)], idx_v)
        pltpu.sync_copy(tab_hbm.at[p0, pl.ds(0, SEG_LEN[0])],
                        buf_a.at[pl.ds(0, SEG_LEN[0])])

        def pair2_body(j, carry):
            for i in range(2):
                k = j * 2 + i
                p = p0 + k
                f = lax.shift_right_logical(p, 5)
                r = i
                q = i
                d1 = start_seg(p, 1, 1 - r)
                d1.start()
                @pl.when(jnp.logical_and(jnp.bitwise_and(p, 31) == 0, k > 0))
                def _():
                    pltpu.sync_copy(idx_hbm.at[f], idx_v)

                @pl.when(k >= 2)
                def _():
                    out_copy(p, q).wait()

                pass_t(0, r, q)
                d1.wait()
                d2 = start_seg(p, 2, r)
                d2.start()
                dt = tail_copy(p, r)
                dt.start()
                pass_t(1, 1 - r, q)
                d2.wait()
                dt.wait()
                pn = jnp.minimum(p + 1, NROWS - 1)
                d0 = start_seg(pn, 0, 1 - r)
                d0.start()
                pass_t(2, r, q)
                out_copy(p, q).start()
                d0.wait()
            return carry

        lax.fori_loop(0, PAIRS_PER_W // 2, pair2_body, 0)
        out_copy(p0, 0).wait()
        out_copy(p0, 1).wait()

    return gather_k


_sc_gather = _make_sc_gather()


def _mlp_body(xct_ref, cat_ref, w1_ref, b1_ref, g_ref, be_ref,
              w2h_ref, w2c_ref, b2_ref, w3_ref, b3_ref, out_ref):
    h = jnp.dot(w1_ref[...], xct_ref[...],
                preferred_element_type=jnp.float32) + b1_ref[...]
    mu = jnp.mean(h, axis=0, keepdims=True)
    var = jnp.mean((h - mu) * (h - mu), axis=0, keepdims=True)
    h = (h - mu) * lax.rsqrt(var + 1e-5) * g_ref[...] + be_ref[...]
    h = h * (1.0 / (1.0 + jnp.exp(-h)))
    z = (jnp.dot(w2h_ref[...], h, preferred_element_type=jnp.float32)
         + jnp.dot(w2c_ref[...], cat_ref[...],
                   preferred_element_type=jnp.float32)
         + b2_ref[...])
    z = z * (1.0 / (1.0 + jnp.exp(-z)))
    z = jnp.dot(w3_ref[...], z, preferred_element_type=jnp.float32) + b3_ref[...]
    out_ref[...] = z * (1.0 / (1.0 + jnp.exp(-z)))


_BCOL = 2048


def _mlp_call(xct, catt, w1, b1, ln_g, ln_b, w2h, w2c, b2, w3, b3):
    grid = (B // _BCOL,)
    full = lambda *shape: pl.BlockSpec(shape, lambda i: (0,) * len(shape))
    return pl.pallas_call(
        _mlp_body,
        grid=grid,
        in_specs=[
            pl.BlockSpec((NUM_CONT, _BCOL), lambda i: (0, i)),
            pl.BlockSpec((NROWS, _BCOL), lambda i: (0, i)),
            full(H_CONT, NUM_CONT),
            full(H_CONT, 1),
            full(H_CONT, 1),
            full(H_CONT, 1),
            full(H1, H_CONT),
            full(H1, NROWS),
            full(H1, 1),
            full(H2, H1),
            full(H2, 1),
        ],
        out_specs=pl.BlockSpec((H2, _BCOL), lambda i: (0, i)),
        out_shape=jax.ShapeDtypeStruct((H2, B), jnp.float32),
    )(xct, catt, w1, b1, ln_g, ln_b, w2h, w2c, b2, w3, b3)


def kernel(x_cont, x_cat, tables, W1, b1, ln_g, ln_b, W2, b2, W3, b3):
    tab_t = jnp.transpose(tables, (0, 2, 1)).reshape(NROWS, VOCAB)
    tail_t = jnp.pad(tab_t[:, TAIL_OFF:], ((0, 0), (0, TAIL_PAD - TAIL_LEN)))
    idx_t = x_cat.T
    cat_t = _sc_gather(idx_t, tab_t, tail_t)
    out_t = _mlp_call(
        x_cont.T, cat_t,
        W1, b1.reshape(-1, 1), ln_g.reshape(-1, 1), ln_b.reshape(-1, 1),
        W2[:, :H_CONT], W2[:, H_CONT:], b2.reshape(-1, 1),
        W3, b3.reshape(-1, 1),
    )
    return out_t.T
